# Initial kernel scaffold; baseline (speedup 1.0000x reference)
import jax
import jax.numpy as jnp
from jax.experimental import pallas as pl


def kernel(edge_index, h, weights_first, weights_middle, weights_last, lin_W, lin_b, bn_gamma, bn_beta, concat_W, concat_b, bnh_gamma, bnh_beta):
    raise NotImplementedError("write your pallas kernel here")


# speedup vs baseline: 12.7277x; 12.7277x over previous
import functools

import jax
import jax.numpy as jnp
from jax import lax
from jax.experimental import pallas as pl
from jax.experimental.pallas import tpu as pltpu
from jax.experimental.pallas import tpu_sc as plsc

_N = 10000
_E = 320000
_D = 128

_NC = 2
_NS = 16
_NW = _NC * _NS
_EPW = _E // _NW
_CH = 80
_NCHK = _EPW // _CH
_NP = 10240
_RPT = _NP // _NS
_ZR = 128


_MESH = plsc.VectorSubcoreMesh(core_axis_name="c", subcore_axis_name="s")


def _zero_fill(buf, rows, value=0.0):
    val = jnp.full((16,), value, jnp.float32)

    def zrow(i, carry):
        for l in range(_D // 16):
            buf[i, pl.ds(l * 16, 16)] = val
        return carry

    lax.fori_loop(0, rows, zrow, 0)


def _zero_spmem(agg, zbuf, s):
    _zero_fill(zbuf, _ZR)
    for t in range(_RPT // _ZR):
        pltpu.sync_copy(zbuf, agg.at[pl.ds(s * _RPT + t * _ZR, _ZR)])


def _flush_spmem(agg, out_hbm, s):
    for t in range(_RPT // _ZR):
        off = s * _RPT + t * _ZR
        pltpu.sync_copy(agg.at[pl.ds(off, _ZR)], out_hbm.at[pl.ds(off, _ZR)])


_CHP = 64
_NCH = 160
_EPWP = _CHP * _NCH
_K = 4
_J = 3
_SLAB = 32
_NSLAB = _NCH // _SLAB


@functools.partial(
    pl.kernel,
    out_type=[
        jax.ShapeDtypeStruct((_NC, _NP, _D), jnp.float32),
        jax.ShapeDtypeStruct((_NC, _NP), jnp.float32),
    ],
    mesh=_MESH,
    scratch_types=[
        pltpu.VMEM((2 * _SLAB, _CHP), jnp.int32),
        pltpu.VMEM((_K, _CHP), jnp.int32),
        pltpu.VMEM((_K, _CHP), jnp.int32),
        pltpu.VMEM((_K, _CHP, _D), jnp.float32),
        pltpu.VMEM((_CHP,), jnp.float32),
        pltpu.VMEM((_ZR,), jnp.float32),
        pltpu.VMEM_SHARED((_NP, _D), jnp.float32),
        pltpu.VMEM_SHARED((_NP,), jnp.float32),
        pltpu.SemaphoreType.DMA,
        pltpu.SemaphoreType.DMA,
        pltpu.SemaphoreType.DMA,
        pltpu.SemaphoreType.DMA,
    ],
)
def _gcn(x_hbm, pk_hbm, out_hbm, dout_hbm, pkbuf, srcbuf, dstbuf, gbuf,
         ones_v, zb1, agg, deg, gsem, ssem, dsem, psem):
    c = lax.axis_index("c")
    s = lax.axis_index("s")
    w = c * _NS + s
    val1 = jnp.ones((16,), jnp.float32)
    zro = jnp.zeros((16,), jnp.float32)
    for l in range(_CHP // 16):
        ones_v[pl.ds(l * 16, 16)] = val1
    for l in range(_ZR // 16):
        zb1[pl.ds(l * 16, 16)] = zro
    _zero_fill(gbuf.at[0], _CHP)
    for t in range(_RPT // _CHP):
        pltpu.sync_copy(gbuf.at[0], agg.at[pl.ds(s * _RPT + t * _CHP, _CHP)])
    for t in range(_RPT // _ZR):
        pltpu.sync_copy(zb1, deg.at[pl.ds(s * _RPT + t * _ZR, _ZR)])
    plsc.subcore_barrier()

    mask16 = jnp.full((16,), 0xFFFF, jnp.int32)

    def _fire_slab(sg):
        pltpu.async_copy(
            pk_hbm.at[w, sg],
            pkbuf.at[pl.ds(lax.rem(sg, 2) * _SLAB, _SLAB)], psem)

    def _wait_slab(sg):
        pltpu.make_async_copy(
            pk_hbm.at[w, sg],
            pkbuf.at[pl.ds(lax.rem(sg, 2) * _SLAB, _SLAB)], psem).wait()

    def _unpack(b, m):
        r = lax.rem(b, 2 * _SLAB)
        for l in range(_CHP // 16):
            p = pkbuf[r, pl.ds(l * 16, 16)]
            srcbuf[m, pl.ds(l * 16, 16)] = jnp.bitwise_and(p, mask16)
            dstbuf[m, pl.ds(l * 16, 16)] = lax.shift_right_logical(p, 16)

    def _fire_gather(b, m):
        pltpu.async_copy(x_hbm.at[srcbuf.at[m]], gbuf.at[m], gsem)

    def _wait_gather(b, m):
        pltpu.make_async_copy(x_hbm.at[srcbuf.at[m]], gbuf.at[m], gsem).wait()

    def _fire_scatter(b, m):
        pltpu.async_copy(gbuf.at[m], agg.at[dstbuf.at[m]], ssem, add=True)
        pltpu.async_copy(ones_v, deg.at[dstbuf.at[m]], dsem, add=True)

    def _wait_scatter(b, m):
        pltpu.make_async_copy(gbuf.at[m], agg.at[dstbuf.at[m]], ssem).wait()
        pltpu.make_async_copy(ones_v, deg.at[dstbuf.at[m]], dsem).wait()

    _fire_slab(0)
    _wait_slab(0)
    _fire_slab(1)
    for b0 in range(_J):
        _unpack(b0, b0)
        _fire_gather(b0, b0)

    def body(b, carry):
        m = lax.rem(b, _K)
        _wait_gather(b, m)
        _fire_scatter(b, m)

        @pl.when(b + _J < _NCH)
        def _():
            m2 = lax.rem(b + _J, _K)

            @pl.when(lax.rem(b + _J, _SLAB) == 0)
            def _():
                sg = lax.div(b + _J, _SLAB)
                _wait_slab(sg)

                @pl.when(sg + 1 < _NSLAB)
                def _():
                    _fire_slab(sg + 1)

            @pl.when(b >= _K - _J)
            def _():
                _wait_scatter(b - (_K - _J), m2)

            _unpack(b + _J, m2)
            _fire_gather(b + _J, m2)

        return carry

    lax.fori_loop(0, _NCH, body, 0)
    for b in range(_NCH - _K, _NCH):
        _wait_scatter(b, b % _K)

    plsc.subcore_barrier()
    _flush_spmem(agg, out_hbm.at[c], s)
    for t in range(_RPT // _ZR):
        off = s * _RPT + t * _ZR
        pltpu.sync_copy(deg.at[pl.ds(off, _ZR)], dout_hbm.at[c, pl.ds(off, _ZR)])


def _mix(x, gx, hin, getW, g, b):
    acc = None
    for k, xk in enumerate((x, gx, hin)):
        y = jnp.dot(xk, getW(k), preferred_element_type=jnp.float32)
        mu = jnp.mean(y, axis=0, keepdims=True)
        yc = y - mu
        var = jnp.mean(yc * yc, axis=0, keepdims=True)
        z = yc * lax.rsqrt(var + 1e-5)
        t = jnp.maximum(z * g[k] + b[k], 0.0)
        acc = t if acc is None else acc + t
    return acc


_TC_PARAMS = pltpu.CompilerParams(vmem_limit_bytes=63 * 1024 * 1024)


def _norm_partials(gp_ref, rdeg):
    return (gp_ref[0, :_N] + gp_ref[1, :_N]) * rdeg


def _tc1_body(gh_ref, degp_ref, h_ref, w_ref, g_ref, b_ref, s1_ref, ghn_ref,
              rdeg_ref):
    dsum = degp_ref[:, :_N]
    ones2 = jnp.ones((2, 1), jnp.float32)
    deg = lax.dot_general(dsum, ones2, (((0,), (0,)), ((), ())),
                          preferred_element_type=jnp.float32)
    rdeg = 1.0 / jnp.maximum(deg, 1.0)
    rdeg_ref[...] = rdeg
    ghn = _norm_partials(gh_ref, rdeg)
    ghn_ref[...] = ghn
    hh = h_ref[...]
    s1_ref[...] = _mix(hh, ghn, hh, lambda k: w_ref[k], g_ref[...], b_ref[...])


def _tc1(gh, degp, h, w, g, b):
    return pl.pallas_call(
        _tc1_body,
        out_shape=[
            jax.ShapeDtypeStruct((_N, _D), jnp.float32),
            jax.ShapeDtypeStruct((_N, _D), jnp.float32),
            jax.ShapeDtypeStruct((_N, 1), jnp.float32),
        ],
        compiler_params=_TC_PARAMS,
    )(gh, degp, h, w, g, b)


def _tcnorm_body(gp_ref, rdeg_ref, gx_ref):
    gx_ref[...] = _norm_partials(gp_ref, rdeg_ref[...])


def _tcnorm(gp, rdeg):
    return pl.pallas_call(
        _tcnorm_body,
        out_shape=jax.ShapeDtypeStruct((_N, _D), jnp.float32),
        compiler_params=_TC_PARAMS,
    )(gp, rdeg)


def _tcmix1_body(x_ref, gx_ref, h_ref, w_ref, g_ref, b_ref, out_ref):
    out_ref[...] = _mix(x_ref[...], gx_ref[...], h_ref[...],
                        lambda k: w_ref[k], g_ref[...], b_ref[...])


def _tcmix1(x, gx, hin, w, g, b):
    return pl.pallas_call(
        _tcmix1_body,
        out_shape=jax.ShapeDtypeStruct((_N, _D), jnp.float32),
        compiler_params=_TC_PARAMS,
    )(x, gx, hin, w, g, b)


def _tcmix1p_body(x_ref, gp_ref, rdeg_ref, h_ref, w_ref, g_ref, b_ref,
                  out_ref):
    gx = _norm_partials(gp_ref, rdeg_ref[...])
    out_ref[...] = _mix(x_ref[...], gx, h_ref[...], lambda k: w_ref[k],
                        g_ref[...], b_ref[...])


def _tcmix1p(x, gp, rdeg, hin, w, g, b):
    return pl.pallas_call(
        _tcmix1p_body,
        out_shape=jax.ShapeDtypeStruct((_N, _D), jnp.float32),
        compiler_params=_TC_PARAMS,
    )(x, gp, rdeg, hin, w, g, b)


def _tcmix1pa_body(x_ref, gp_ref, rdeg_ref, h_ref, w_ref, g_ref, b_ref,
                   add_ref, out_ref):
    gx = _norm_partials(gp_ref, rdeg_ref[...])
    out_ref[...] = add_ref[...] + _mix(x_ref[...], gx, h_ref[...],
                                       lambda k: w_ref[k], g_ref[...],
                                       b_ref[...])


def _tcmix1pa(x, gp, rdeg, hin, w, g, b, addin):
    return pl.pallas_call(
        _tcmix1pa_body,
        out_shape=jax.ShapeDtypeStruct((_N, _D), jnp.float32),
        compiler_params=_TC_PARAMS,
    )(x, gp, rdeg, hin, w, g, b, addin)


def _tcfinal_body(h_ref, m0_ref, m1_ref, gp_ref, rdeg_ref, w_ref, g_ref,
                  b_ref, sa_ref, cw_ref, bg_ref, bb_ref, out_ref):
    gx = _norm_partials(gp_ref, rdeg_ref[...])
    m1 = m1_ref[...]
    hh = h_ref[...]
    s = sa_ref[...] + _mix(m1, gx, hh, lambda k: w_ref[k], g_ref[...],
                           b_ref[...])
    y = (jnp.dot(m0_ref[...], cw_ref[0:_D], preferred_element_type=jnp.float32)
         + jnp.dot(m1, cw_ref[_D:2 * _D],
                   preferred_element_type=jnp.float32)
         + jnp.dot(s, cw_ref[2 * _D:3 * _D],
                   preferred_element_type=jnp.float32))
    mu = jnp.mean(y, axis=0, keepdims=True)
    yc = y - mu
    var = jnp.mean(yc * yc, axis=0, keepdims=True)
    z = yc * lax.rsqrt(var + 1e-5)
    out_ref[...] = hh + jnp.maximum(z * bg_ref[...] + bb_ref[...], 0.0)


def _tcfinal(h, m0, m1, gp, rdeg, w, g, b, sa, cw, bg, bb):
    return pl.pallas_call(
        _tcfinal_body,
        out_shape=jax.ShapeDtypeStruct((_N, _D), jnp.float32),
        compiler_params=_TC_PARAMS,
    )(h, m0, m1, gp, rdeg, w, g, b, sa, cw, bg, bb)


def kernel(edge_index, h, weights_first, weights_middle, weights_last, lin_W,
           lin_b, bn_gamma, bn_beta, concat_W, concat_b, bnh_gamma, bnh_beta):
    pad = _NW * _EPWP - _E
    pad_iota = jnp.arange(pad, dtype=jnp.int32)
    pad_src = pad_iota % _N
    pad_dst = _N + pad_iota % (_NP - _N)
    packed = jnp.concatenate([
        edge_index[0] | (edge_index[1] << 16),
        pad_src | (pad_dst << 16),
    ]).reshape(_NW, _NSLAB, _SLAB, _CHP)
    w_all = jnp.concatenate([weights_first, weights_middle, weights_last], 0)
    g_eff = bn_gamma * w_all[:, :, None]
    b_eff = bn_beta * w_all[:, :, None]

    ghp, degp = _gcn(h, packed)
    s1, ghn, rdeg = _tc1(ghp, degp, h, lin_W[0], g_eff[0], b_eff[0])
    gs1p, _ = _gcn(s1, packed)
    sx = _tcmix1(h, ghn, h, lin_W[1], g_eff[1], b_eff[1])
    s2 = _tcmix1pa(s1, gs1p, rdeg, h, lin_W[2], g_eff[2], b_eff[2], sx)
    m0 = _tcmix1p(s1, gs1p, rdeg, h, lin_W[3], g_eff[3], b_eff[3])
    gs2p, _ = _gcn(s2, packed)
    m1 = _tcmix1p(s2, gs2p, rdeg, h, lin_W[4], g_eff[4], b_eff[4])
    gm0p, _ = _gcn(m0, packed)
    gm1p, _ = _gcn(m1, packed)
    gm0n = _tcnorm(gm0p, rdeg)
    sa = _tcmix1(m0, gm0n, h, lin_W[5], g_eff[5], b_eff[5])
    return _tcfinal(h, m0, m1, gm1p, rdeg, lin_W[6], g_eff[6], b_eff[6], sa,
                    concat_W, bnh_gamma, bnh_beta)
